# baseline (speedup 1.0000x reference)
import functools
import math

import jax
import jax.numpy as jnp
from jax import lax
from jax.experimental import pallas as pl
from jax.experimental.pallas import tpu as pltpu
from jax.experimental.pallas import tpu_sc as plsc

B = 16384
L = 50
D = 64
LANES = 16
NC, NS = 2, 16
NW = NC * NS
BPW = B // NW
HALF = BPW // 2
DT, DI = D // 8, 8
BT, BI = B // 128, 128
NCH = 2 * L
NBUF = 4
SCALE = math.sqrt(D)


def _build():
  mesh = plsc.VectorSubcoreMesh(core_axis_name="c", subcore_axis_name="s")

  @functools.partial(
      pl.kernel,
      out_type=jax.ShapeDtypeStruct((L, DT, BT, DI, BI), jnp.float32),
      mesh=mesh,
      compiler_params=pltpu.CompilerParams(
          use_tc_tiling_on_sc=False, needs_layout_passes=False),
      scratch_types=[
          pltpu.VMEM((L, BPW), jnp.int32),
          pltpu.VMEM((NBUF, HALF, D), jnp.float32),
          pltpu.VMEM((2, 2, DT, DI, BI + 1), jnp.float32),
          pltpu.SemaphoreType.DMA,
          pltpu.SemaphoreType.DMA,
          pltpu.SemaphoreType.DMA,
          pltpu.SemaphoreType.DMA,
          pltpu.SemaphoreType.DMA,
          pltpu.SemaphoreType.DMA,
      ],
  )
  def embed(xt_hbm, table_hbm, out_hbm, idx_all, rows, tiles,
            g0, g1, g2, g3, s0, s1):
    wid = lax.axis_index("s") * NC + lax.axis_index("c")
    gsem = (g0, g1, g2, g3)
    ssem = (s0, s1)
    iota = lax.iota(jnp.int32, LANES)

    def fire(l, h, k):
      for j in range(2):
        idx_sl = idx_all.at[l, pl.ds(HALF * h + 128 * j, 128)]
        pltpu.async_copy(table_hbm.at[idx_sl],
                         rows.at[k, pl.ds(128 * j, 128)], gsem[k])

    def drain(k):
      for j in range(2):
        idx_sl = idx_all.at[0, pl.ds(128 * j, 128)]
        pltpu.make_async_copy(table_hbm.at[idx_sl],
                              rows.at[k, pl.ds(128 * j, 128)],
                              gsem[k]).wait()

    dtv = [(16 * c + iota) >> 3 for c in range(D // LANES)]
    div = iota & 7

    def fill_store(l, h, k, have_prev):
      r = rows.at[k]
      ob = k % 2
      o = tiles.at[ob]

      @pl.when(have_prev)
      def _():
        for btl in range(2):
          pltpu.make_async_copy(o.at[btl, :, :, pl.ds(0, BI)],
                                out_hbm.at[0, :, btl], ssem[ob]).wait()

      for btl in range(2):
        bv = jnp.full((LANES,), btl, jnp.int32)

        @pl.loop(0, 128, unroll=8)
        def _(rr):
          row = 128 * btl + rr
          biv = jnp.full((LANES,), rr, jnp.int32)
          for c in range(D // LANES):
            v = r[row, pl.ds(16 * c, 16)] * SCALE
            plsc.store_scatter(o, [bv, dtv[c], div, biv], v)

      bt0 = 4 * wid + 2 * h
      for btl in range(2):
        pltpu.async_copy(o.at[btl, :, :, pl.ds(0, BI)],
                         out_hbm.at[l, :, bt0 + btl], ssem[ob])

    pltpu.sync_copy(xt_hbm.at[:, pl.ds(BPW * wid, BPW)], idx_all)
    for c in range(NBUF - 1):
      fire(c // 2, c % 2, c)

    @pl.loop(0, NCH // NBUF)
    def _(i):
      for k in range(NBUF):
        lk = NBUF // 2 * i + k // 2
        hk = k % 2
        nc = NBUF * i + k + NBUF - 1
        nk = (k + NBUF - 1) % NBUF

        @pl.when(nc < NCH)
        def _():
          fire(nc // 2, nc % 2, nk)

        drain(k)
        fill_store(lk, hk, k, (i > 0) if k < 2 else (i >= 0))

    for ob in range(2):
      for btl in range(2):
        pltpu.make_async_copy(tiles.at[ob, btl, :, :, pl.ds(0, BI)],
                              out_hbm.at[0, :, btl], ssem[ob]).wait()

  return embed


@jax.jit
def kernel(x, table):
  xt = jnp.swapaxes(x.astype(jnp.int32), 0, 1)
  out5 = _build()(xt, table)
  return out5.transpose(2, 4, 0, 1, 3).reshape(B, L, D)


# speedup vs baseline: 1.0057x; 1.0057x over previous
import functools
import math

import jax
import jax.numpy as jnp
from jax import lax
from jax.experimental import pallas as pl
from jax.experimental.pallas import tpu as pltpu
from jax.experimental.pallas import tpu_sc as plsc

B = 16384
L = 50
D = 64
LANES = 16
NC, NS = 2, 16
NW = NC * NS
BPW = B // NW
HALF = BPW // 2
DT, DI = D // 8, 8
BT, BI = B // 128, 128
NCH = 2 * L
NBUF = 2
SCALE = math.sqrt(D)


def _build():
  mesh = plsc.VectorSubcoreMesh(core_axis_name="c", subcore_axis_name="s")

  @functools.partial(
      pl.kernel,
      out_type=jax.ShapeDtypeStruct((L, DT, BT, DI, BI), jnp.float32),
      mesh=mesh,
      compiler_params=pltpu.CompilerParams(
          use_tc_tiling_on_sc=False, needs_layout_passes=False),
      scratch_types=[
          pltpu.VMEM((L, BPW), jnp.int32),
          pltpu.VMEM((NBUF, HALF, D), jnp.float32),
          pltpu.VMEM((2, 2, DT, DI, BI + 1), jnp.float32),
          pltpu.SemaphoreType.DMA,
          pltpu.SemaphoreType.DMA,
          pltpu.SemaphoreType.DMA,
          pltpu.SemaphoreType.DMA,
          pltpu.SemaphoreType.DMA,
          pltpu.SemaphoreType.DMA,
      ],
  )
  def embed(xt_hbm, table_hbm, out_hbm, idx_all, rows, tiles,
            g0, g1, g2, g3, s0, s1):
    wid = lax.axis_index("s") * NC + lax.axis_index("c")
    gsem = (g0, g1, g2, g3)
    ssem = (s0, s1)
    iota = lax.iota(jnp.int32, LANES)

    def fire(l, h, k):
      for j in range(2):
        idx_sl = idx_all.at[l, pl.ds(HALF * h + 128 * j, 128)]
        pltpu.async_copy(table_hbm.at[idx_sl],
                         rows.at[k, pl.ds(128 * j, 128)], gsem[k])

    def drain(k):
      for j in range(2):
        idx_sl = idx_all.at[0, pl.ds(128 * j, 128)]
        pltpu.make_async_copy(table_hbm.at[idx_sl],
                              rows.at[k, pl.ds(128 * j, 128)],
                              gsem[k]).wait()

    dtv = [(16 * c + iota) >> 3 for c in range(D // LANES)]
    div = iota & 7

    def fill_store(l, h, k, have_prev):
      r = rows.at[k]
      ob = k % 2
      o = tiles.at[ob]

      @pl.when(have_prev)
      def _():
        for btl in range(2):
          pltpu.make_async_copy(o.at[btl, :, :, pl.ds(0, BI)],
                                out_hbm.at[0, :, btl], ssem[ob]).wait()

      for btl in range(2):
        bv = jnp.full((LANES,), btl, jnp.int32)

        @pl.loop(0, 128, unroll=4)
        def _(rr):
          row = 128 * btl + rr
          biv = jnp.full((LANES,), rr, jnp.int32)
          for c in range(D // LANES):
            v = r[row, pl.ds(16 * c, 16)] * SCALE
            plsc.store_scatter(o, [bv, dtv[c], div, biv], v)

      bt0 = 4 * wid + 2 * h
      for btl in range(2):
        pltpu.async_copy(o.at[btl, :, :, pl.ds(0, BI)],
                         out_hbm.at[l, :, bt0 + btl], ssem[ob])

    pltpu.sync_copy(xt_hbm.at[:, pl.ds(BPW * wid, BPW)], idx_all)
    for c in range(NBUF - 1):
      fire(c // 2, c % 2, c)

    @pl.loop(0, NCH // NBUF)
    def _(i):
      for k in range(NBUF):
        lk = NBUF // 2 * i + k // 2
        hk = k % 2
        nc = NBUF * i + k + NBUF - 1
        nk = (k + NBUF - 1) % NBUF

        @pl.when(nc < NCH)
        def _():
          fire(nc // 2, nc % 2, nk)

        drain(k)
        fill_store(lk, hk, k, (i > 0) if k < 2 else (i >= 0))

    for ob in range(2):
      for btl in range(2):
        pltpu.make_async_copy(tiles.at[ob, btl, :, :, pl.ds(0, BI)],
                              out_hbm.at[0, :, btl], ssem[ob]).wait()

  return embed


@jax.jit
def kernel(x, table):
  xt = jnp.swapaxes(x.astype(jnp.int32), 0, 1)
  out5 = _build()(xt, table)
  return out5.transpose(2, 4, 0, 1, 3).reshape(B, L, D)


# speedup vs baseline: 1.0078x; 1.0021x over previous
import functools
import math

import jax
import jax.numpy as jnp
from jax import lax
from jax.experimental import pallas as pl
from jax.experimental.pallas import tpu as pltpu
from jax.experimental.pallas import tpu_sc as plsc

B = 16384
L = 50
D = 64
LANES = 16
NC, NS = 2, 16
NW = NC * NS
BPW = B // NW
HALF = BPW // 2
DT, DI = D // 8, 8
BT, BI = B // 128, 128
NCH = 2 * L
NBUF = 2
SCALE = math.sqrt(D)


def _build():
  mesh = plsc.VectorSubcoreMesh(core_axis_name="c", subcore_axis_name="s")

  @functools.partial(
      pl.kernel,
      out_type=jax.ShapeDtypeStruct((L, DT, BT, DI, BI), jnp.float32),
      mesh=mesh,
      compiler_params=pltpu.CompilerParams(
          use_tc_tiling_on_sc=False, needs_layout_passes=False),
      scratch_types=[
          pltpu.VMEM((L, BPW), jnp.int32),
          pltpu.VMEM((NBUF, HALF, D), jnp.float32),
          pltpu.VMEM((2, 2, DT, DI, BI + 1), jnp.float32),
          pltpu.SemaphoreType.DMA,
          pltpu.SemaphoreType.DMA,
          pltpu.SemaphoreType.DMA,
          pltpu.SemaphoreType.DMA,
      ],
  )
  def embed(xt_hbm, table_hbm, out_hbm, idx_all, rows, tiles,
            g0, g1, s0, s1):
    wid = lax.axis_index("s") * NC + lax.axis_index("c")
    gsem = (g0, g1)
    ssem = (s0, s1)
    iota = lax.iota(jnp.int32, LANES)

    def fire(l, h, k):
      for j in range(2):
        idx_sl = idx_all.at[l, pl.ds(HALF * h + 128 * j, 128)]
        pltpu.async_copy(table_hbm.at[idx_sl],
                         rows.at[k, pl.ds(128 * j, 128)], gsem[k])

    def drain(k):
      for j in range(2):
        idx_sl = idx_all.at[0, pl.ds(128 * j, 128)]
        pltpu.make_async_copy(table_hbm.at[idx_sl],
                              rows.at[k, pl.ds(128 * j, 128)],
                              gsem[k]).wait()

    dtv = [(16 * c + iota) >> 3 for c in range(D // LANES)]
    div = iota & 7

    def fill_store(l, h, k, have_prev):
      r = rows.at[k]
      ob = k % 2
      o = tiles.at[ob]

      @pl.when(have_prev)
      def _():
        for btl in range(2):
          pltpu.make_async_copy(o.at[btl, :, :, pl.ds(0, BI)],
                                out_hbm.at[0, :, btl], ssem[ob]).wait()

      for btl in range(2):
        bv = jnp.full((LANES,), btl, jnp.int32)

        @pl.loop(0, 128, unroll=4)
        def _(rr):
          row = 128 * btl + rr
          biv = jnp.full((LANES,), rr, jnp.int32)
          for c in range(D // LANES):
            v = r[row, pl.ds(16 * c, 16)] * SCALE
            plsc.store_scatter(o, [bv, dtv[c], div, biv], v)

      bt0 = 4 * wid + 2 * h
      for btl in range(2):
        pltpu.async_copy(o.at[btl, :, :, pl.ds(0, BI)],
                         out_hbm.at[l, :, bt0 + btl], ssem[ob])

    pltpu.sync_copy(xt_hbm.at[:, pl.ds(BPW * wid, BPW)], idx_all)
    for c in range(NBUF - 1):
      fire(c // 2, c % 2, c)

    @pl.loop(0, NCH // NBUF)
    def _(i):
      for k in range(NBUF):
        lk = NBUF // 2 * i + k // 2
        hk = k % 2
        nc = NBUF * i + k + NBUF - 1
        nk = (k + NBUF - 1) % NBUF

        @pl.when(nc < NCH)
        def _():
          fire(nc // 2, nc % 2, nk)

        drain(k)
        fill_store(lk, hk, k, (i > 0) if k < 2 else (i >= 0))

    for ob in range(2):
      for btl in range(2):
        pltpu.make_async_copy(tiles.at[ob, btl, :, :, pl.ds(0, BI)],
                              out_hbm.at[0, :, btl], ssem[ob]).wait()

  return embed


@jax.jit
def kernel(x, table):
  xt = jnp.swapaxes(x.astype(jnp.int32), 0, 1)
  out5 = _build()(xt, table)
  return out5.transpose(2, 4, 0, 1, 3).reshape(B, L, D)
